# baseline (speedup 1.0000x reference)
import functools

import jax
import jax.numpy as jnp
from jax import lax
from jax.experimental import pallas as pl
from jax.experimental.pallas import tpu as pltpu
from jax.experimental.pallas import tpu_sc as plsc

N = 10000
E = 320000
D = 128
NPAD = 10240
EPAD = 327680
NW = 32
EPW = EPAD // NW
STEP = 128
NSTEPS = EPW // STEP
RPT = NPAD // 16
NCH = 8
CCOL = 16
NB = 2048
GRID = NPAD // NB

_mesh = plsc.VectorSubcoreMesh(core_axis_name="c", subcore_axis_name="s")


def _k1_body(h0_ref, h1_ref, wd_ref, wa_ref, wcp_ref, svals_ref, zc_ref):
    h0 = h0_ref[...]
    wsum = wcp_ref[0:4, 0:4] + wcp_ref[0:4, 4:8]
    zs = []
    sds = []
    for j in range(4):
        zj = lax.dot_general(h0, wd_ref[j], (((1,), (1,)), ((), ())),
                             preferred_element_type=jnp.float32)
        zs.append(zj)
        sds.append(lax.dot_general(zj, wa_ref[j:j + 1, D:2 * D],
                                   (((1,), (1,)), ((), ())),
                                   preferred_element_type=jnp.float32))
    for o in range(4):
        acc = zs[0] * wsum[o:o + 1, 0:1]
        for j in range(1, 4):
            acc = acc + zs[j] * wsum[o:o + 1, j:j + 1]
        zc_ref[:, o, :] = acc
    ss0 = lax.dot_general(h1_ref[:, 0, :], wa_ref[:, 0:D],
                          (((1,), (1,)), ((), ())),
                          preferred_element_type=jnp.float32)
    ss1 = lax.dot_general(h1_ref[:, 1, :], wa_ref[:, 0:D],
                          (((1,), (1,)), ((), ())),
                          preferred_element_type=jnp.float32)
    sval = jnp.concatenate([ss0, ss1] + sds + [jnp.zeros((NB, 4), jnp.float32)],
                           axis=1)
    svals_ref[...] = sval


_k1 = pl.pallas_call(
    _k1_body,
    grid=(GRID,),
    in_specs=[
        pl.BlockSpec((NB, D), lambda i: (i, 0)),
        pl.BlockSpec((NB, 2, D), lambda i: (i, 0, 0)),
        pl.BlockSpec((4, D, D), lambda i: (0, 0, 0)),
        pl.BlockSpec((4, 2 * D), lambda i: (0, 0)),
        pl.BlockSpec((8, 128), lambda i: (0, 0)),
    ],
    out_specs=[
        pl.BlockSpec((NB, 16), lambda i: (i, 0)),
        pl.BlockSpec((NB, 4, D), lambda i: (i, 0, 0)),
    ],
    out_shape=[
        jax.ShapeDtypeStruct((NPAD, 16), jnp.float32),
        jax.ShapeDtypeStruct((NPAD, 4, D), jnp.float32),
    ],
)


@functools.partial(
    pl.kernel,
    out_type=[
        jax.ShapeDtypeStruct((2 * NPAD, 16), jnp.float32),
        jax.ShapeDtypeStruct((EPAD, 16), jnp.float32),
    ],
    mesh=_mesh,
    compiler_params=pltpu.CompilerParams(needs_layout_passes=False,
                                         use_tc_tiling_on_sc=False),
    scratch_types=[
        pltpu.VMEM_SHARED((NPAD, 16), jnp.float32),
        pltpu.VMEM((STEP,), jnp.int32),
        pltpu.VMEM((STEP,), jnp.int32),
        pltpu.VMEM((STEP, 16), jnp.float32),
        pltpu.VMEM((STEP, 16), jnp.float32),
        pltpu.VMEM((STEP, 16), jnp.float32),
        pltpu.SemaphoreType.DMA,
        pltpu.SemaphoreType.DMA,
    ],
)
def _passb(src_hbm, dst_hbm, svals_hbm, accout_hbm, p16_hbm,
           acc, srcv, dstv, ssrc, sdst, updv, sem1, sem2):
    cid = lax.axis_index("c")
    sid = lax.axis_index("s")
    wid = cid * 16 + sid

    def zrow(r, carry):
        updv[r, :] = jnp.zeros((16,), jnp.float32)
        return carry
    lax.fori_loop(0, STEP, zrow, 0)

    def zacc(t, carry):
        pltpu.sync_copy(updv, acc.at[pl.ds(sid * RPT + t * STEP, STEP)])
        return carry
    lax.fori_loop(0, RPT // STEP, zacc, 0)
    plsc.subcore_barrier()

    def step(t, carry):
        base = wid * EPW + t * STEP
        pltpu.sync_copy(src_hbm.at[pl.ds(base, STEP)], srcv)
        pltpu.sync_copy(dst_hbm.at[pl.ds(base, STEP)], dstv)
        cp1 = pltpu.async_copy(svals_hbm.at[srcv], ssrc, sem1)
        cp2 = pltpu.async_copy(svals_hbm.at[dstv], sdst, sem2)
        cp1.wait()
        cp2.wait()
        ones = jnp.ones((16,), jnp.float32)
        for g in range(8):
            rows = lax.iota(jnp.int32, 16) + g * 16
            for k in range(8):
                j = k % 4
                colk = jnp.full((16,), k, jnp.int32)
                ps = plsc.load_gather(ssrc, [rows, colk])
                pd = plsc.load_gather(sdst, [rows, jnp.full((16,), 8 + j, jnp.int32)])
                a = ps + pd
                e = jnp.where(a > 0, a, a * 0.01)
                p = jnp.exp(jnp.minimum(e, 60.0))
                plsc.store_scatter(updv, [rows, colk], p)
            plsc.store_scatter(updv, [rows, jnp.full((16,), 8, jnp.int32)], ones)
        pltpu.sync_copy(updv, acc.at[dstv], add=True)
        pltpu.sync_copy(updv, p16_hbm.at[pl.ds(base, STEP)])
        return carry
    lax.fori_loop(0, NSTEPS, step, 0)
    plsc.subcore_barrier()

    rowbase = cid * NPAD + sid * RPT
    pltpu.sync_copy(acc.at[pl.ds(sid * RPT, RPT)],
                    accout_hbm.at[pl.ds(rowbase, RPT)])


def _k1b_body(acc_ref, rden_ref):
    d = acc_ref[0] + acc_ref[1]
    rden_ref[...] = jnp.where(d > 0, 1.0 / jnp.maximum(d, 1e-30), 0.0)


_k1b = pl.pallas_call(
    _k1b_body,
    grid=(GRID,),
    in_specs=[pl.BlockSpec((2, NB, 16), lambda i: (0, i, 0))],
    out_specs=pl.BlockSpec((NB, 16), lambda i: (i, 0)),
    out_shape=jax.ShapeDtypeStruct((NPAD, 16), jnp.float32),
)


@functools.partial(
    pl.kernel,
    out_type=jax.ShapeDtypeStruct((EPAD, 16), jnp.float32),
    mesh=_mesh,
    compiler_params=pltpu.CompilerParams(needs_layout_passes=False,
                                         use_tc_tiling_on_sc=False),
    scratch_types=[
        pltpu.VMEM((NSTEPS, STEP), jnp.int32),
        pltpu.VMEM((STEP,), jnp.int32),
        pltpu.VMEM((STEP,), jnp.int32),
        pltpu.VMEM((STEP, 16), jnp.float32),
        pltpu.VMEM((STEP, 16), jnp.float32),
        pltpu.VMEM((STEP, 16), jnp.float32),
        pltpu.VMEM((STEP, 16), jnp.float32),
        pltpu.VMEM((STEP, 16), jnp.float32),
        pltpu.VMEM((STEP, 16), jnp.float32),
        pltpu.VMEM((32,), jnp.float32),
        pltpu.SemaphoreType.DMA,
        pltpu.SemaphoreType.DMA,
        pltpu.SemaphoreType.DMA,
        pltpu.SemaphoreType.DMA,
    ],
)
def _passc0(dst3_hbm, p16_hbm, rden_hbm, wc_hbm, beta16_hbm,
            dstall, dstva, dstvb, p16a, p16b, dena, denb, betaa, betab, wcv,
            semg0, semg1, semw0, semw1):
    cid = lax.axis_index("c")
    sid = lax.axis_index("s")
    wid = cid * 16 + sid
    ebase = wid * EPW

    pltpu.sync_copy(wc_hbm, wcv)
    pltpu.sync_copy(dst3_hbm.at[pl.ds(wid * NSTEPS, NSTEPS)], dstall)

    def _fill(dst_ref, tab_ref, t):
        for g in range(8):
            dst_ref[pl.ds(g * 16, 16)] = tab_ref[t, pl.ds(g * 16, 16)]

    wca = wcv[pl.ds(0, 16)]
    wcb = wcv[pl.ds(16, 16)]
    wcs = [wca[k] for k in range(16)] + [wcb[k] for k in range(16)]

    dstvs = (dstva, dstvb)
    p16s = (p16a, p16b)
    dens = (dena, denb)
    betas = (betaa, betab)
    semg = (semg0, semg1)
    semw = (semw0, semw1)

    def issue_bload(t, s):
        _fill(dstvs[s], dstall, t)
        pltpu.async_copy(p16_hbm.at[pl.ds(ebase + t * STEP, STEP)],
                         p16s[s], semg[s])
        pltpu.async_copy(rden_hbm.at[dstvs[s]], dens[s], semg[s])

    def wait_bload(s):
        pltpu.make_async_copy(p16_hbm.at[pl.ds(0, STEP)], p16s[s], semg[s]).wait()
        pltpu.make_async_copy(rden_hbm.at[dstvs[s]], dens[s], semg[s]).wait()

    def bpair(g, carry):
        for s in range(1):
            t = g
            issue_bload(t, s)
            wait_bload(s)
            for gg in range(8):
                rows = lax.iota(jnp.int32, 16) + gg * 16
                al = []
                for k in range(8):
                    colk = jnp.full((16,), k, jnp.int32)
                    p = plsc.load_gather(p16s[s], [rows, colk])
                    r = plsc.load_gather(dens[s], [rows, colk])
                    al.append(p * r)
                for i in range(2):
                    for o in range(4):
                        b = al[i * 4] * wcs[o * 8 + i * 4]
                        for j in range(1, 4):
                            b = b + al[i * 4 + j] * wcs[o * 8 + i * 4 + j]
                        plsc.store_scatter(
                            betas[s],
                            [rows, jnp.full((16,), i * 4 + o, jnp.int32)], b)
            pltpu.sync_copy(betas[s],
                            beta16_hbm.at[pl.ds(ebase + t * STEP, STEP)])
        return carry
    lax.fori_loop(0, NSTEPS, bpair, 0)


@functools.partial(
    pl.kernel,
    out_type=jax.ShapeDtypeStruct((NCH, 2 * NPAD, 4 * CCOL), jnp.float32),
    mesh=_mesh,
    compiler_params=pltpu.CompilerParams(needs_layout_passes=False,
                                         use_tc_tiling_on_sc=False),
    scratch_types=[
        pltpu.VMEM_SHARED((NPAD, 4 * CCOL), jnp.float32),
        pltpu.VMEM((NSTEPS, STEP), jnp.int32),
        pltpu.VMEM((NSTEPS, STEP), jnp.int32),
        pltpu.VMEM((STEP,), jnp.int32),
        pltpu.VMEM((STEP,), jnp.int32),
        pltpu.VMEM((STEP,), jnp.int32),
        pltpu.VMEM((STEP,), jnp.int32),
        pltpu.VMEM((STEP, 16), jnp.float32),
        pltpu.VMEM((STEP, 16), jnp.float32),
        pltpu.VMEM((STEP, 2 * CCOL), jnp.float32),
        pltpu.VMEM((STEP, 2 * CCOL), jnp.float32),
        pltpu.VMEM((STEP, 4 * CCOL), jnp.float32),
        pltpu.VMEM((STEP, 4 * CCOL), jnp.float32),
        pltpu.SemaphoreType.DMA,
        pltpu.SemaphoreType.DMA,
        pltpu.SemaphoreType.DMA,
        pltpu.SemaphoreType.DMA,
    ],
)
def _passc1(src3_hbm, dst3_hbm, beta16_hbm,
            h1r0_hbm, h1r1_hbm, h1r2_hbm, h1r3_hbm,
            h1r4_hbm, h1r5_hbm, h1r6_hbm, h1r7_hbm,
            scat_hbm,
            acc, srcall, dstall, srcva, srcvb, dstva, dstvb,
            betaa, betab, h1ga, h1gb, upda, updb,
            semg0, semg1, sems0, sems1):
    cid = lax.axis_index("c")
    sid = lax.axis_index("s")
    wid = cid * 16 + sid
    ebase = wid * EPW

    pltpu.sync_copy(src3_hbm.at[pl.ds(wid * NSTEPS, NSTEPS)], srcall)
    pltpu.sync_copy(dst3_hbm.at[pl.ds(wid * NSTEPS, NSTEPS)], dstall)

    def _fill(dst_ref, tab_ref, t):
        for g in range(8):
            dst_ref[pl.ds(g * 16, 16)] = tab_ref[t, pl.ds(g * 16, 16)]

    srcvs = (srcva, srcvb)
    dstvs = (dstva, dstvb)
    betas = (betaa, betab)
    h1gs = (h1ga, h1gb)
    upds = (upda, updb)
    semg = (semg0, semg1)
    sems = (sems0, sems1)

    def issue_cload(h1r, t, s):
        _fill(srcvs[s], srcall, t)
        pltpu.async_copy(h1r.at[srcvs[s]], h1gs[s], semg[s])
        pltpu.async_copy(beta16_hbm.at[pl.ds(ebase + t * STEP, STEP)],
                         betas[s], semg[s])

    def wait_cload(h1r, s):
        pltpu.make_async_copy(h1r.at[srcvs[s]], h1gs[s], semg[s]).wait()
        pltpu.make_async_copy(beta16_hbm.at[pl.ds(0, STEP)], betas[s], semg[s]).wait()

    h1rs = (h1r0_hbm, h1r1_hbm, h1r2_hbm, h1r3_hbm,
            h1r4_hbm, h1r5_hbm, h1r6_hbm, h1r7_hbm)
    for c4 in range(NCH):
        h1r = h1rs[c4]

        def zrow(r, carry):
            for q in range(4):
                upda[r, pl.ds(q * 16, 16)] = jnp.zeros((16,), jnp.float32)
            return carry
        lax.fori_loop(0, STEP, zrow, 0)

        def zacc(t, carry):
            pltpu.sync_copy(upda, acc.at[pl.ds(sid * RPT + t * STEP, STEP)])
            return carry
        lax.fori_loop(0, RPT // STEP, zacc, 0)
        plsc.subcore_barrier()

        def cpair(g, carry):
            for s in range(1):
                t = g
                issue_cload(h1r, t, s)
                wait_cload(h1r, s)

                h1v = h1gs[s]
                bev = betas[s]
                upv = upds[s]

                def edge(e2, carry2):
                    h0c = h1v[e2, pl.ds(0, 16)]
                    h1c = h1v[e2, pl.ds(16, 16)]
                    bv = bev[e2, :]
                    for o in range(4):
                        upv[e2, pl.ds(o * 16, 16)] = h0c * bv[o] + h1c * bv[4 + o]
                    return carry2
                lax.fori_loop(0, STEP, edge, 0)

                _fill(dstvs[s], dstall, t)
                pltpu.sync_copy(upds[s], acc.at[dstvs[s]], add=True)
            return carry
        lax.fori_loop(0, NSTEPS, cpair, 0)
        plsc.subcore_barrier()

        rowbase = cid * NPAD + sid * RPT

        def dump(t, carry):
            pltpu.sync_copy(acc.at[pl.ds(sid * RPT + t * STEP, STEP)],
                            scat_hbm.at[c4, pl.ds(rowbase + t * STEP, STEP)])
            return carry
        lax.fori_loop(0, RPT // STEP, dump, 0)


def _k2_body(scat_ref, zc_ref, acc_ref, out_ref):
    deg = acc_ref[0, :, 8:9] + acc_ref[1, :, 8:9]
    mask = deg > 0
    inv = 1.0 / jnp.maximum(deg, 1.0)
    s = [scat_ref[c, 0] + scat_ref[c, 1] for c in range(NCH)]
    for o in range(4):
        so = jnp.concatenate(
            [s[c][:, o * CCOL:(o + 1) * CCOL] for c in range(NCH)],
            axis=1)
        v = zc_ref[:, o, :] * inv + so
        out_ref[:, o, :] = jnp.where(mask, v, 0.0)


NB2 = 512

_k2 = pl.pallas_call(
    _k2_body,
    grid=(NPAD // NB2,),
    in_specs=[
        pl.BlockSpec((NCH, 2, NB2, 4 * CCOL), lambda i: (0, 0, i, 0)),
        pl.BlockSpec((NB2, 4, 128), lambda i: (i, 0, 0)),
        pl.BlockSpec((2, NB2, 16), lambda i: (0, i, 0)),
    ],
    out_specs=pl.BlockSpec((NB2, 4, 128), lambda i: (i, 0, 0)),
    out_shape=jax.ShapeDtypeStruct((NPAD, 4, D), jnp.float32),
)


def kernel(edge_index, hier_1, hier_0, W_dst, W_attn, Wn):
    src = edge_index[0]
    dst = edge_index[1]
    pad_ids = (jnp.arange(EPAD - E, dtype=jnp.int32) % 32) + N
    srcp = jnp.concatenate([src, pad_ids])
    dstp = jnp.concatenate([dst, pad_ids])
    h0p = jnp.pad(hier_0, ((0, NPAD - N), (0, 0)))
    h1p = jnp.pad(hier_1, ((0, NPAD - N), (0, 0), (0, 0)))

    wc = jnp.maximum(Wn, 0.0)
    wc = wc / jnp.sum(wc, axis=0, keepdims=True)
    wcp = jnp.zeros((8, 128), jnp.float32).at[0:4, 0:8].set(wc)
    wc32 = wc.reshape(-1)

    h1r = jnp.transpose(h1p.reshape(NPAD, 2, NCH, CCOL),
                        (2, 0, 1, 3)).reshape(NCH, NPAD, 2 * CCOL)

    svals, zc = _k1(h0p, h1p, W_dst, W_attn, wcp)
    accp, p16 = _passb(srcp, dstp, svals)
    rden = _k1b(accp.reshape(2, NPAD, 16))
    src2 = srcp.reshape(NW * NSTEPS, STEP)
    dst2 = dstp.reshape(NW * NSTEPS, STEP)
    beta16 = _passc0(dst2, p16, rden, wc32)
    scat = _passc1(src2, dst2, beta16,
                   h1r[0], h1r[1], h1r[2], h1r[3],
                   h1r[4], h1r[5], h1r[6], h1r[7])
    out = _k2(scat.reshape(NCH, 2, NPAD, 4 * CCOL), zc,
              accp.reshape(2, NPAD, 16))
    return out[:N]


# speedup vs baseline: 1.5458x; 1.5458x over previous
import functools

import jax
import jax.numpy as jnp
from jax import lax
from jax.experimental import pallas as pl
from jax.experimental.pallas import tpu as pltpu
from jax.experimental.pallas import tpu_sc as plsc

N = 10000
E = 320000
D = 128
NPAD = 10240
EPAD = 327680
NW = 32
EPW = EPAD // NW
STEP = 128
NSTEPS = EPW // STEP
RPT = NPAD // 16
NCH = 8
CCOL = 16
NB = 2048
GRID = NPAD // NB

_mesh = plsc.VectorSubcoreMesh(core_axis_name="c", subcore_axis_name="s")


def _k1_body(h0_ref, h1_ref, wd_ref, wa_ref, wcp_ref, svals_ref, zc_ref):
    h0 = h0_ref[...]
    wsum = wcp_ref[0:4, 0:4] + wcp_ref[0:4, 4:8]
    zs = []
    sds = []
    for j in range(4):
        zj = lax.dot_general(h0, wd_ref[j], (((1,), (1,)), ((), ())),
                             preferred_element_type=jnp.float32)
        zs.append(zj)
        sds.append(lax.dot_general(zj, wa_ref[j:j + 1, D:2 * D],
                                   (((1,), (1,)), ((), ())),
                                   preferred_element_type=jnp.float32))
    for o in range(4):
        acc = zs[0] * wsum[o:o + 1, 0:1]
        for j in range(1, 4):
            acc = acc + zs[j] * wsum[o:o + 1, j:j + 1]
        zc_ref[:, o, :] = acc
    ss0 = lax.dot_general(h1_ref[:, 0, :], wa_ref[:, 0:D],
                          (((1,), (1,)), ((), ())),
                          preferred_element_type=jnp.float32)
    ss1 = lax.dot_general(h1_ref[:, 1, :], wa_ref[:, 0:D],
                          (((1,), (1,)), ((), ())),
                          preferred_element_type=jnp.float32)
    sval = jnp.concatenate([ss0, ss1] + sds + [jnp.zeros((NB, 4), jnp.float32)],
                           axis=1)
    svals_ref[...] = sval


_k1 = pl.pallas_call(
    _k1_body,
    grid=(GRID,),
    in_specs=[
        pl.BlockSpec((NB, D), lambda i: (i, 0)),
        pl.BlockSpec((NB, 2, D), lambda i: (i, 0, 0)),
        pl.BlockSpec((4, D, D), lambda i: (0, 0, 0)),
        pl.BlockSpec((4, 2 * D), lambda i: (0, 0)),
        pl.BlockSpec((8, 128), lambda i: (0, 0)),
    ],
    out_specs=[
        pl.BlockSpec((NB, 16), lambda i: (i, 0)),
        pl.BlockSpec((NB, 4, D), lambda i: (i, 0, 0)),
    ],
    out_shape=[
        jax.ShapeDtypeStruct((NPAD, 16), jnp.float32),
        jax.ShapeDtypeStruct((NPAD, 4, D), jnp.float32),
    ],
)


@functools.partial(
    pl.kernel,
    out_type=[
        jax.ShapeDtypeStruct((2 * NPAD, 16), jnp.float32),
        jax.ShapeDtypeStruct((EPAD, 16), jnp.float32),
    ],
    mesh=_mesh,
    compiler_params=pltpu.CompilerParams(needs_layout_passes=False,
                                         use_tc_tiling_on_sc=False),
    scratch_types=[
        pltpu.VMEM_SHARED((NPAD, 16), jnp.float32),
        pltpu.VMEM((STEP,), jnp.int32),
        pltpu.VMEM((STEP,), jnp.int32),
        pltpu.VMEM((STEP, 16), jnp.float32),
        pltpu.VMEM((STEP, 16), jnp.float32),
        pltpu.VMEM((STEP, 16), jnp.float32),
        pltpu.SemaphoreType.DMA,
        pltpu.SemaphoreType.DMA,
    ],
)
def _passb(src_hbm, dst_hbm, svals_hbm, accout_hbm, p16_hbm,
           acc, srcv, dstv, ssrc, sdst, updv, sem1, sem2):
    cid = lax.axis_index("c")
    sid = lax.axis_index("s")
    wid = cid * 16 + sid

    def zrow(r, carry):
        updv[r, :] = jnp.zeros((16,), jnp.float32)
        return carry
    lax.fori_loop(0, STEP, zrow, 0)

    def zacc(t, carry):
        pltpu.sync_copy(updv, acc.at[pl.ds(sid * RPT + t * STEP, STEP)])
        return carry
    lax.fori_loop(0, RPT // STEP, zacc, 0)
    plsc.subcore_barrier()

    def step(t, carry):
        base = wid * EPW + t * STEP
        pltpu.sync_copy(src_hbm.at[pl.ds(base, STEP)], srcv)
        pltpu.sync_copy(dst_hbm.at[pl.ds(base, STEP)], dstv)
        cp1 = pltpu.async_copy(svals_hbm.at[srcv], ssrc, sem1)
        cp2 = pltpu.async_copy(svals_hbm.at[dstv], sdst, sem2)
        cp1.wait()
        cp2.wait()
        ones = jnp.ones((16,), jnp.float32)
        for g in range(8):
            rows = lax.iota(jnp.int32, 16) + g * 16
            for k in range(8):
                j = k % 4
                colk = jnp.full((16,), k, jnp.int32)
                ps = plsc.load_gather(ssrc, [rows, colk])
                pd = plsc.load_gather(sdst, [rows, jnp.full((16,), 8 + j, jnp.int32)])
                a = ps + pd
                e = jnp.where(a > 0, a, a * 0.01)
                p = jnp.exp(jnp.minimum(e, 60.0))
                plsc.store_scatter(updv, [rows, colk], p)
            plsc.store_scatter(updv, [rows, jnp.full((16,), 8, jnp.int32)], ones)
        pltpu.sync_copy(updv, acc.at[dstv], add=True)
        pltpu.sync_copy(updv, p16_hbm.at[pl.ds(base, STEP)])
        return carry
    lax.fori_loop(0, NSTEPS, step, 0)
    plsc.subcore_barrier()

    rowbase = cid * NPAD + sid * RPT
    pltpu.sync_copy(acc.at[pl.ds(sid * RPT, RPT)],
                    accout_hbm.at[pl.ds(rowbase, RPT)])


def _k1b_body(acc_ref, rden_ref):
    d = acc_ref[0] + acc_ref[1]
    rden_ref[...] = jnp.where(d > 0, 1.0 / jnp.maximum(d, 1e-30), 0.0)


_k1b = pl.pallas_call(
    _k1b_body,
    grid=(GRID,),
    in_specs=[pl.BlockSpec((2, NB, 16), lambda i: (0, i, 0))],
    out_specs=pl.BlockSpec((NB, 16), lambda i: (i, 0)),
    out_shape=jax.ShapeDtypeStruct((NPAD, 16), jnp.float32),
)


@functools.partial(
    pl.kernel,
    out_type=jax.ShapeDtypeStruct((EPAD, 16), jnp.float32),
    mesh=_mesh,
    compiler_params=pltpu.CompilerParams(needs_layout_passes=False,
                                         use_tc_tiling_on_sc=False),
    scratch_types=[
        pltpu.VMEM((NSTEPS, STEP), jnp.int32),
        pltpu.VMEM((STEP,), jnp.int32),
        pltpu.VMEM((STEP,), jnp.int32),
        pltpu.VMEM((STEP, 16), jnp.float32),
        pltpu.VMEM((STEP, 16), jnp.float32),
        pltpu.VMEM((STEP, 16), jnp.float32),
        pltpu.VMEM((STEP, 16), jnp.float32),
        pltpu.VMEM((STEP, 16), jnp.float32),
        pltpu.VMEM((STEP, 16), jnp.float32),
        pltpu.VMEM((32,), jnp.float32),
        pltpu.SemaphoreType.DMA,
        pltpu.SemaphoreType.DMA,
        pltpu.SemaphoreType.DMA,
        pltpu.SemaphoreType.DMA,
    ],
)
def _passc0(dst3_hbm, p16_hbm, rden_hbm, wc_hbm, beta16_hbm,
            dstall, dstva, dstvb, p16a, p16b, dena, denb, betaa, betab, wcv,
            semg0, semg1, semw0, semw1):
    cid = lax.axis_index("c")
    sid = lax.axis_index("s")
    wid = cid * 16 + sid
    ebase = wid * EPW

    pltpu.sync_copy(wc_hbm, wcv)
    pltpu.sync_copy(dst3_hbm.at[pl.ds(wid * NSTEPS, NSTEPS)], dstall)

    def _fill(dst_ref, tab_ref, t):
        for g in range(8):
            dst_ref[pl.ds(g * 16, 16)] = tab_ref[t, pl.ds(g * 16, 16)]

    wca = wcv[pl.ds(0, 16)]
    wcb = wcv[pl.ds(16, 16)]
    wcs = [wca[k] for k in range(16)] + [wcb[k] for k in range(16)]

    dstvs = (dstva, dstvb)
    p16s = (p16a, p16b)
    dens = (dena, denb)
    betas = (betaa, betab)
    semg = (semg0, semg1)
    semw = (semw0, semw1)

    def issue_bload(t, s):
        _fill(dstvs[s], dstall, t)
        pltpu.async_copy(p16_hbm.at[pl.ds(ebase + t * STEP, STEP)],
                         p16s[s], semg[s])
        pltpu.async_copy(rden_hbm.at[dstvs[s]], dens[s], semg[s])

    def wait_bload(s):
        pltpu.make_async_copy(p16_hbm.at[pl.ds(0, STEP)], p16s[s], semg[s]).wait()
        pltpu.make_async_copy(rden_hbm.at[dstvs[s]], dens[s], semg[s]).wait()

    issue_bload(0, 0)

    def bpair(g, carry):
        for s in range(2):
            t = g * 2 + s

            @pl.when(t + 1 < NSTEPS)
            def _():
                issue_bload(t + 1, 1 - s)
            wait_bload(s)

            @pl.when(t >= 2)
            def _():
                pltpu.make_async_copy(betas[s], beta16_hbm.at[pl.ds(0, STEP)],
                                      semw[s]).wait()
            for gg in range(8):
                rows = lax.iota(jnp.int32, 16) + gg * 16
                al = []
                for k in range(8):
                    colk = jnp.full((16,), k, jnp.int32)
                    p = plsc.load_gather(p16s[s], [rows, colk])
                    r = plsc.load_gather(dens[s], [rows, colk])
                    al.append(p * r)
                for i in range(2):
                    for o in range(4):
                        b = al[i * 4] * wcs[o * 8 + i * 4]
                        for j in range(1, 4):
                            b = b + al[i * 4 + j] * wcs[o * 8 + i * 4 + j]
                        plsc.store_scatter(
                            betas[s],
                            [rows, jnp.full((16,), i * 4 + o, jnp.int32)], b)
            pltpu.async_copy(betas[s],
                             beta16_hbm.at[pl.ds(ebase + t * STEP, STEP)],
                             semw[s])
        return carry
    lax.fori_loop(0, NSTEPS // 2, bpair, 0)
    pltpu.make_async_copy(betaa, beta16_hbm.at[pl.ds(0, STEP)], semw[0]).wait()
    pltpu.make_async_copy(betab, beta16_hbm.at[pl.ds(0, STEP)], semw[1]).wait()


@functools.partial(
    pl.kernel,
    out_type=jax.ShapeDtypeStruct((NCH, 2 * NPAD, 4 * CCOL), jnp.float32),
    mesh=_mesh,
    compiler_params=pltpu.CompilerParams(needs_layout_passes=False,
                                         use_tc_tiling_on_sc=False),
    scratch_types=[
        pltpu.VMEM_SHARED((NPAD, 4 * CCOL), jnp.float32),
        pltpu.VMEM((NSTEPS, STEP), jnp.int32),
        pltpu.VMEM((NSTEPS, STEP), jnp.int32),
        pltpu.VMEM((STEP,), jnp.int32),
        pltpu.VMEM((STEP,), jnp.int32),
        pltpu.VMEM((STEP,), jnp.int32),
        pltpu.VMEM((STEP,), jnp.int32),
        pltpu.VMEM((STEP, 16), jnp.float32),
        pltpu.VMEM((STEP, 16), jnp.float32),
        pltpu.VMEM((STEP, 2 * CCOL), jnp.float32),
        pltpu.VMEM((STEP, 2 * CCOL), jnp.float32),
        pltpu.VMEM((STEP, 4 * CCOL), jnp.float32),
        pltpu.VMEM((STEP, 4 * CCOL), jnp.float32),
        pltpu.SemaphoreType.DMA,
        pltpu.SemaphoreType.DMA,
        pltpu.SemaphoreType.DMA,
        pltpu.SemaphoreType.DMA,
    ],
)
def _passc1(src3_hbm, dst3_hbm, beta16_hbm,
            h1r0_hbm, h1r1_hbm, h1r2_hbm, h1r3_hbm,
            h1r4_hbm, h1r5_hbm, h1r6_hbm, h1r7_hbm,
            scat_hbm,
            acc, srcall, dstall, srcva, srcvb, dstva, dstvb,
            betaa, betab, h1ga, h1gb, upda, updb,
            semg0, semg1, sems0, sems1):
    cid = lax.axis_index("c")
    sid = lax.axis_index("s")
    wid = cid * 16 + sid
    ebase = wid * EPW

    pltpu.sync_copy(src3_hbm.at[pl.ds(wid * NSTEPS, NSTEPS)], srcall)
    pltpu.sync_copy(dst3_hbm.at[pl.ds(wid * NSTEPS, NSTEPS)], dstall)

    def _fill(dst_ref, tab_ref, t):
        for g in range(8):
            dst_ref[pl.ds(g * 16, 16)] = tab_ref[t, pl.ds(g * 16, 16)]

    srcvs = (srcva, srcvb)
    dstvs = (dstva, dstvb)
    betas = (betaa, betab)
    h1gs = (h1ga, h1gb)
    upds = (upda, updb)
    semg = (semg0, semg1)
    sems = (sems0, sems1)

    def issue_cload(h1r, t, s):
        _fill(srcvs[s], srcall, t)
        pltpu.async_copy(h1r.at[srcvs[s]], h1gs[s], semg[s])
        pltpu.async_copy(beta16_hbm.at[pl.ds(ebase + t * STEP, STEP)],
                         betas[s], semg[s])

    def wait_cload(h1r, s):
        pltpu.make_async_copy(h1r.at[srcvs[s]], h1gs[s], semg[s]).wait()
        pltpu.make_async_copy(beta16_hbm.at[pl.ds(0, STEP)], betas[s], semg[s]).wait()

    h1rs = (h1r0_hbm, h1r1_hbm, h1r2_hbm, h1r3_hbm,
            h1r4_hbm, h1r5_hbm, h1r6_hbm, h1r7_hbm)
    for c4 in range(NCH):
        h1r = h1rs[c4]

        def zrow(r, carry):
            for q in range(4):
                upda[r, pl.ds(q * 16, 16)] = jnp.zeros((16,), jnp.float32)
            return carry
        lax.fori_loop(0, STEP, zrow, 0)

        def zacc(t, carry):
            pltpu.sync_copy(upda, acc.at[pl.ds(sid * RPT + t * STEP, STEP)])
            return carry
        lax.fori_loop(0, RPT // STEP, zacc, 0)
        plsc.subcore_barrier()

        issue_cload(h1r, 0, 0)

        def cpair(g, carry):
            for s in range(2):
                t = g * 2 + s

                @pl.when(t + 1 < NSTEPS)
                def _():
                    issue_cload(h1r, t + 1, 1 - s)
                wait_cload(h1r, s)

                @pl.when(t >= 2)
                def _():
                    pltpu.make_async_copy(upds[s], acc.at[dstvs[s]],
                                          sems[s]).wait()

                h1v = h1gs[s]
                bev = betas[s]
                upv = upds[s]

                def edge(e2, carry2):
                    h0c = h1v[e2, pl.ds(0, 16)]
                    h1c = h1v[e2, pl.ds(16, 16)]
                    bv = bev[e2, :]
                    for o in range(4):
                        upv[e2, pl.ds(o * 16, 16)] = h0c * bv[o] + h1c * bv[4 + o]
                    return carry2
                lax.fori_loop(0, STEP, edge, 0)

                _fill(dstvs[s], dstall, t)
                pltpu.async_copy(upds[s], acc.at[dstvs[s]], sems[s], add=True)
            return carry
        lax.fori_loop(0, NSTEPS // 2, cpair, 0)
        pltpu.make_async_copy(upda, acc.at[dstva], sems[0]).wait()
        pltpu.make_async_copy(updb, acc.at[dstvb], sems[1]).wait()
        plsc.subcore_barrier()

        rowbase = cid * NPAD + sid * RPT

        def dump(t, carry):
            pltpu.sync_copy(acc.at[pl.ds(sid * RPT + t * STEP, STEP)],
                            scat_hbm.at[c4, pl.ds(rowbase + t * STEP, STEP)])
            return carry
        lax.fori_loop(0, RPT // STEP, dump, 0)


def _k2_body(scat_ref, zc_ref, acc_ref, out_ref):
    deg = acc_ref[0, :, 8:9] + acc_ref[1, :, 8:9]
    mask = deg > 0
    inv = 1.0 / jnp.maximum(deg, 1.0)
    s = [scat_ref[c, 0] + scat_ref[c, 1] for c in range(NCH)]
    for o in range(4):
        so = jnp.concatenate(
            [s[c][:, o * CCOL:(o + 1) * CCOL] for c in range(NCH)],
            axis=1)
        v = zc_ref[:, o, :] * inv + so
        out_ref[:, o, :] = jnp.where(mask, v, 0.0)


NB2 = 512

_k2 = pl.pallas_call(
    _k2_body,
    grid=(NPAD // NB2,),
    in_specs=[
        pl.BlockSpec((NCH, 2, NB2, 4 * CCOL), lambda i: (0, 0, i, 0)),
        pl.BlockSpec((NB2, 4, 128), lambda i: (i, 0, 0)),
        pl.BlockSpec((2, NB2, 16), lambda i: (0, i, 0)),
    ],
    out_specs=pl.BlockSpec((NB2, 4, 128), lambda i: (i, 0, 0)),
    out_shape=jax.ShapeDtypeStruct((NPAD, 4, D), jnp.float32),
)


def kernel(edge_index, hier_1, hier_0, W_dst, W_attn, Wn):
    src = edge_index[0]
    dst = edge_index[1]
    pad_ids = (jnp.arange(EPAD - E, dtype=jnp.int32) % 32) + N
    srcp = jnp.concatenate([src, pad_ids])
    dstp = jnp.concatenate([dst, pad_ids])
    h0p = jnp.pad(hier_0, ((0, NPAD - N), (0, 0)))
    h1p = jnp.pad(hier_1, ((0, NPAD - N), (0, 0), (0, 0)))

    wc = jnp.maximum(Wn, 0.0)
    wc = wc / jnp.sum(wc, axis=0, keepdims=True)
    wcp = jnp.zeros((8, 128), jnp.float32).at[0:4, 0:8].set(wc)
    wc32 = wc.reshape(-1)

    h1r = jnp.transpose(h1p.reshape(NPAD, 2, NCH, CCOL),
                        (2, 0, 1, 3)).reshape(NCH, NPAD, 2 * CCOL)

    svals, zc = _k1(h0p, h1p, W_dst, W_attn, wcp)
    accp, p16 = _passb(srcp, dstp, svals)
    rden = _k1b(accp.reshape(2, NPAD, 16))
    src2 = srcp.reshape(NW * NSTEPS, STEP)
    dst2 = dstp.reshape(NW * NSTEPS, STEP)
    beta16 = _passc0(dst2, p16, rden, wc32)
    scat = _passc1(src2, dst2, beta16,
                   h1r[0], h1r[1], h1r[2], h1r[3],
                   h1r[4], h1r[5], h1r[6], h1r[7])
    out = _k2(scat.reshape(NCH, 2, NPAD, 4 * CCOL), zc,
              accp.reshape(2, NPAD, 16))
    return out[:N]
